# baseline (speedup 1.0000x reference)
import jax
import jax.numpy as jnp
from jax.experimental import pallas as pl
from jax.experimental.pallas import tpu as pltpu

NUM_EXPERTS = 64
TOP_K = 2
BT2 = 2048


def _top2(logits_t):
    iota = jax.lax.broadcasted_iota(jnp.int32, logits_t.shape, 0)
    neg_inf = jnp.float32(float("-inf"))
    m0 = jnp.max(logits_t, axis=0, keepdims=True)
    i0 = jnp.min(jnp.where(logits_t == m0, iota, NUM_EXPERTS), axis=0,
                 keepdims=True)
    masked = jnp.where(iota == i0, neg_inf, logits_t)
    m1 = jnp.max(masked, axis=0, keepdims=True)
    i1 = jnp.min(jnp.where(masked == m1, iota, NUM_EXPERTS), axis=0,
                 keepdims=True)
    e = jnp.exp(m1 - m0)
    s = 1.0 / (1.0 + e)
    return (jnp.concatenate([s, e * s], axis=0),
            jnp.concatenate([i0, i1], axis=0))


def _router_kernel(xlo_ref, xhi_ref, w_ref, packed_ref,
                   wlo_ref, whi_ref, elo_ref, ehi_ref):
    x_lo = xlo_ref[0]
    x_hi = xhi_ref[0]
    w = w_ref[...]

    logits_lo = jax.lax.dot_general(
        x_lo, w, (((1,), (1,)), ((), ())),
        preferred_element_type=jnp.float32)
    logits_hi = jax.lax.dot_general(
        x_hi, w, (((1,), (1,)), ((), ())),
        preferred_element_type=jnp.float32)
    packed_ref[0] = jnp.concatenate([logits_lo, logits_hi], axis=1)

    lt_lo = jax.lax.dot_general(
        w, x_lo, (((1,), (1,)), ((), ())),
        preferred_element_type=jnp.float32)
    lt_hi = jax.lax.dot_general(
        w, x_hi, (((1,), (1,)), ((), ())),
        preferred_element_type=jnp.float32)

    wlo_ref[0], elo_ref[0] = _top2(lt_lo)
    whi_ref[0], ehi_ref[0] = _top2(lt_hi)


def kernel(hidden_states, gate_w):
    b, seq, hidden = hidden_states.shape
    half = seq // 2
    sb = half // BT2
    grid = (b * sb,)

    small = jax.ShapeDtypeStruct((b, TOP_K, half), jnp.float32)
    small_i = jax.ShapeDtypeStruct((b, TOP_K, half), jnp.int32)
    small_spec = pl.BlockSpec((1, TOP_K, BT2), lambda i: (i // sb, 0, i % sb))

    packed, w_lo, w_hi, e_lo, e_hi = pl.pallas_call(
        _router_kernel,
        grid=grid,
        in_specs=[
            pl.BlockSpec((1, BT2, hidden), lambda i: (i // sb, i % sb, 0)),
            pl.BlockSpec((1, BT2, hidden),
                         lambda i: (i // sb, i % sb + sb, 0)),
            pl.BlockSpec((NUM_EXPERTS, hidden), lambda i: (0, 0)),
        ],
        out_specs=[
            pl.BlockSpec((1, BT2, 2 * NUM_EXPERTS),
                         lambda i: (i // sb, i % sb, 0)),
            small_spec, small_spec, small_spec, small_spec,
        ],
        out_shape=[
            jax.ShapeDtypeStruct((b, half, 2 * NUM_EXPERTS), jnp.float32),
            small, small, small_i, small_i,
        ],
        compiler_params=pltpu.CompilerParams(
            dimension_semantics=("arbitrary",),
        ),
    )(hidden_states, hidden_states, gate_w)

    logits = jnp.concatenate(
        [packed[..., :NUM_EXPERTS], packed[..., NUM_EXPERTS:]], axis=1)
    weights = jnp.concatenate([w_lo, w_hi], axis=2).swapaxes(1, 2)
    experts = jnp.concatenate([e_lo, e_hi], axis=2).swapaxes(1, 2)
    return weights, experts, logits
